# baseline (speedup 1.0000x reference)
import functools

import jax
import jax.numpy as jnp
from jax import lax
from jax.experimental import pallas as pl
from jax.experimental.pallas import tpu as pltpu
from jax.experimental.pallas import tpu_sc as plsc

VOCAB = 100000
DIM = 128
B = 1024
L = 200

_INFO = plsc.get_sparse_core_info()
_NC = _INFO.num_cores
_NS = _INFO.num_subcores
_NW = _NC * _NS

_TOTAL = B * L
_PER_W = _TOTAL // _NW
_CHUNK = 128
_NCHUNK = _PER_W // _CHUNK
_NSET = 3


def _gather_body(table_hbm, idx_hbm, out_hbm, idx_v, rows_v, src_v, gsem, osem):
    wid = lax.axis_index("s") * _NC + lax.axis_index("c")
    chunk0 = wid * _NCHUNK

    pltpu.sync_copy(idx_hbm.at[wid], idx_v)

    def gather_start(j, s):
        pltpu.async_copy(
            table_hbm.at[idx_v.at[j]], rows_v.at[s], gsem.at[s]
        )

    def gather_wait(s):
        pltpu.make_async_copy(
            table_hbm.at[idx_v.at[0]], rows_v.at[s], gsem.at[s]
        ).wait()

    def out_start(j, s):
        row_base = (chunk0 + j) * _CHUNK
        pltpu.async_copy(
            src_v.at[s], out_hbm.at[pl.ds(row_base, _CHUNK)], osem.at[s]
        )

    def out_wait(s):
        pltpu.make_async_copy(
            src_v.at[s], out_hbm.at[pl.ds(0, _CHUNK)], osem.at[s]
        ).wait()

    for s in range(_NSET):
        gather_start(s, s)
        out_start(s, s)

    def body(j, carry):
        s = j % _NSET
        gather_wait(s)
        out_wait(s)

        @pl.when(j < _NCHUNK - _NSET)
        def _():
            gather_start(j + _NSET, s)
            out_start(j + _NSET, s)

        return carry

    lax.fori_loop(0, _NCHUNK, body, 0)


@jax.jit
def kernel(word_ids, emb_weight):
    idx3d = word_ids.reshape(_NW, _NCHUNK, _CHUNK).astype(jnp.int32)
    mesh = plsc.VectorSubcoreMesh(core_axis_name="c", subcore_axis_name="s")
    out = pl.kernel(
        _gather_body,
        out_type=jax.ShapeDtypeStruct((_TOTAL, DIM), jnp.float32),
        mesh=mesh,
        scratch_types=[
            pltpu.VMEM((_NCHUNK, _CHUNK), jnp.int32),
            pltpu.VMEM((_NSET, _CHUNK, DIM), jnp.float32),
            pltpu.VMEM((_NSET, _CHUNK, DIM), jnp.float32),
            pltpu.SemaphoreType.DMA((_NSET,)),
            pltpu.SemaphoreType.DMA((_NSET,)),
        ],
    )(emb_weight, idx3d)
    return out.reshape(B, L, DIM)


# speedup vs baseline: 1.0043x; 1.0043x over previous
import jax
import jax.numpy as jnp
from jax import lax
from jax.experimental import pallas as pl
from jax.experimental.pallas import tpu as pltpu
from jax.experimental.pallas import tpu_sc as plsc

VOCAB = 100000
DIM = 128
B = 1024
L = 200

_INFO = plsc.get_sparse_core_info()
_NC = _INFO.num_cores
_NS = _INFO.num_subcores
_NW = _NC * _NS

_TOTAL = B * L
_PER_W = _TOTAL // _NW
_CHUNK = 128
_NCHUNK = _PER_W // _CHUNK
_GRP = 1
_NGRP = _NCHUNK // _GRP
_NSET = 3
_GROWS = _GRP * _CHUNK


def _gather_body(table_hbm, idx_hbm, out_hbm, idx_v, rows_v, sp, gsem, csem, osem):
    cid = lax.axis_index("c")
    sid = lax.axis_index("s")
    wid = sid * _NC + cid
    chunk0 = wid * _NCHUNK

    pltpu.sync_copy(idx_hbm.at[wid], idx_v)

    def gather_start(g, s):
        for c in range(_GRP):
            pltpu.async_copy(
                table_hbm.at[idx_v.at[g * _GRP + c]],
                rows_v.at[s].at[pl.ds(c * _CHUNK, _CHUNK)],
                gsem.at[s],
            )

    def gather_wait(s):
        pltpu.make_async_copy(
            table_hbm.at[idx_v.at[0]], rows_v.at[s], gsem.at[s]
        ).wait()

    def stage_start(s):
        pltpu.async_copy(rows_v.at[s], sp.at[sid].at[s], csem.at[s])

    def stage_wait(s):
        pltpu.make_async_copy(rows_v.at[s], sp.at[sid].at[s], csem.at[s]).wait()

    def out_start(g, s):
        row_base = (chunk0 + g * _GRP) * _CHUNK
        pltpu.async_copy(
            sp.at[sid].at[s], out_hbm.at[pl.ds(row_base, _GROWS)], osem.at[s]
        )

    def out_wait(s):
        pltpu.make_async_copy(
            sp.at[sid].at[0], out_hbm.at[pl.ds(0, _GROWS)], osem.at[s]
        ).wait()

    gather_start(0, 0)
    gather_start(1, 1)

    def body(g, carry):
        s = g % _NSET
        o = (g + 2) % _NSET

        @pl.when(g >= 1)
        def _():
            stage_wait(o)
            out_start(g - 1, o)

        @pl.when(g < _NGRP - 2)
        def _():
            gather_start(g + 2, o)

        gather_wait(s)

        @pl.when(g >= _NSET)
        def _():
            out_wait(s)

        stage_start(s)
        return carry

    lax.fori_loop(0, _NGRP, body, 0)

    last = _NGRP - 1
    stage_wait(last % _NSET)
    out_start(last, last % _NSET)
    out_wait((last - 2) % _NSET)
    out_wait((last - 1) % _NSET)
    out_wait(last % _NSET)


@jax.jit
def kernel(word_ids, emb_weight):
    idx3d = word_ids.reshape(_NW, _NCHUNK, _CHUNK).astype(jnp.int32)
    mesh = plsc.VectorSubcoreMesh(core_axis_name="c", subcore_axis_name="s")
    out = pl.kernel(
        _gather_body,
        out_type=jax.ShapeDtypeStruct((_TOTAL, DIM), jnp.float32),
        mesh=mesh,
        scratch_types=[
            pltpu.VMEM((_NCHUNK, _CHUNK), jnp.int32),
            pltpu.VMEM((_NSET, _GROWS, DIM), jnp.float32),
            pltpu.VMEM_SHARED((_NS, _NSET, _GROWS, DIM), jnp.float32),
            pltpu.SemaphoreType.DMA((_NSET,)),
            pltpu.SemaphoreType.DMA((_NSET,)),
            pltpu.SemaphoreType.DMA((_NSET,)),
        ],
    )(emb_weight, idx3d)
    return out.reshape(B, L, DIM)
